# baseline (speedup 1.0000x reference)
import functools

import jax
import jax.numpy as jnp
from jax.experimental import pallas as pl
from jax.experimental.pallas import tpu as pltpu


def _fused_kernel(x_hbm, perm_ref, wtoe_ref, wtail_ref, bconv_ref,
                  whidT_ref, bhid_ref, wneuT_ref, bneu_ref, out_ref,
                  bufs, sems, x2_ref,
                  *, cin, m, p, s, nfull, tail_start, block_b):
    j = pl.program_id(1)
    g2 = pl.num_programs(1)
    base = (pl.program_id(0) * g2 + j) * block_b
    R = x2_ref.shape[0]
    B = block_b

    @pl.when(j == 0)
    def _():
        for c in range(cin):
            pltpu.make_async_copy(x_hbm.at[pl.ds(base, B), c],
                                  bufs.at[0, c], sems.at[0, c]).start()

    @pl.when(j + 1 < g2)
    def _():
        nbase = base + B
        nslot = (j + 1) % 2
        for c in range(cin):
            pltpu.make_async_copy(x_hbm.at[pl.ds(nbase, B), c],
                                  bufs.at[nslot, c], sems.at[nslot, c]).start()

    slot = j % 2
    for c in range(cin):
        pltpu.make_async_copy(bufs.at[slot, c], bufs.at[slot, c],
                              sems.at[slot, c]).wait()

    dn = (((1,), (1,)), ((), ()))
    xcb = [bufs[slot, c].astype(jnp.bfloat16) for c in range(cin)]
    for grp in range(R // 128):
        y = None
        for c in range(cin):
            pj = perm_ref[(grp * cin + c) * 128:(grp * cin + c + 1) * 128, :]
            d = jax.lax.dot_general(pj, xcb[c], dn,
                                    preferred_element_type=jnp.float32)
            y = d if y is None else y + d
        x2_ref[grp * 128:(grp + 1) * 128, :] = y.astype(jnp.bfloat16)

    wtoe = wtoe_ref[...]
    stride = p * cin
    feat = jnp.full((p * m, B), -1e30, jnp.float32)
    for c in range(nfull):
        xs = x2_ref[pl.ds(c * stride, s), :]
        feat = jnp.maximum(feat, jnp.dot(wtoe, xs,
                                         preferred_element_type=jnp.float32))
    if tail_start is not None:
        xs = x2_ref[pl.ds(tail_start, s), :]
        feat = jnp.maximum(feat, jnp.dot(wtail_ref[...], xs,
                                         preferred_element_type=jnp.float32))

    acc = feat[0:m, :]
    for q in range(1, p):
        acc = jnp.maximum(acc, feat[q * m:(q + 1) * m, :])
    acc = jnp.maximum(acc + bconv_ref[...], 0.0)

    h = jnp.dot(whidT_ref[...], acc,
                preferred_element_type=jnp.float32) + bhid_ref[...]
    h = jnp.maximum(h, 0.0)
    logit = jnp.dot(wneuT_ref[...], h,
                    preferred_element_type=jnp.float32) + bneu_ref[...]
    out_ref[...] = jax.nn.sigmoid(logit)


def _toeplitz(wflat, m, p, s, cin, shift):
    kc = wflat.shape[1]
    colidx = jnp.arange(s)[None, :] - jnp.arange(p)[:, None] * cin - shift
    valid = (colidx >= 0) & (colidx < kc)
    g = wflat[:, jnp.clip(colidx, 0, kc - 1)]
    wt = jnp.where(valid[None], g, 0.0)
    return wt.transpose(1, 0, 2).reshape(p * m, s).astype(jnp.bfloat16)


def kernel(x, wconv, bconv, whid, bhid, wneu, bneu, *, block_b=4096):
    N, cin, L = x.shape
    M, _, K = wconv.shape
    H = whid.shape[1]
    lout = L - K + 1
    P = 128 // M
    S = ((P - 1) * cin + cin * K + 127) // 128 * 128
    R = L * cin
    assert R % 128 == 0 and 128 % cin == 0
    nfull = lout // P
    assert nfull >= 1
    ntail = lout - nfull * P

    if ntail:
        l0 = lout - P
        tail_start = l0 * cin // 16 * 16
        shift = l0 * cin - tail_start
        assert shift + (P - 1) * cin + K * cin <= S
        assert tail_start + S <= R
    else:
        tail_start, shift = None, 0

    npad = max(2 * block_b, (N + 2 * block_b - 1) // (2 * block_b) * (2 * block_b))
    if npad != N:
        x = jnp.pad(x, ((0, npad - N), (0, 0), (0, 0)))

    ngrp = R // 128
    gidx = jnp.arange(ngrp * cin)[:, None]
    jj, cc = gidx // cin, gidx % cin
    r = jnp.arange(128)[None, :]
    tgt = jnp.where((r % cin) == cc, r // cin + (128 // cin) * jj, -1)
    perm = jax.nn.one_hot(tgt, L, dtype=jnp.float32).astype(
        jnp.bfloat16).reshape(ngrp * cin * 128, L)

    wflat = jnp.transpose(wconv.astype(jnp.float32), (0, 2, 1)).reshape(M, K * cin)
    wtoe = _toeplitz(wflat, M, P, S, cin, 0)
    wtail = _toeplitz(wflat, M, P, S, cin, shift) if ntail else wtoe

    bconv2 = bconv.reshape(M, 1).astype(jnp.float32)
    whidT = whid.T.astype(jnp.float32)
    bhid2 = bhid.reshape(H, 1).astype(jnp.float32)
    wneuT = wneu.T.astype(jnp.float32)
    bneu2 = bneu.reshape(1, 1).astype(jnp.float32)

    g2 = npad // (2 * block_b)
    kfn = functools.partial(_fused_kernel, cin=cin, m=M, p=P, s=S,
                            nfull=nfull, tail_start=tail_start,
                            block_b=block_b)
    out = pl.pallas_call(
        kfn,
        out_shape=jax.ShapeDtypeStruct((1, npad), jnp.float32),
        grid_spec=pltpu.PrefetchScalarGridSpec(
            num_scalar_prefetch=0,
            grid=(2, g2),
            in_specs=[
                pl.BlockSpec(memory_space=pl.ANY),
                pl.BlockSpec((ngrp * cin * 128, L), lambda i, j: (0, 0)),
                pl.BlockSpec((P * M, S), lambda i, j: (0, 0)),
                pl.BlockSpec((P * M, S), lambda i, j: (0, 0)),
                pl.BlockSpec((M, 1), lambda i, j: (0, 0)),
                pl.BlockSpec((H, M), lambda i, j: (0, 0)),
                pl.BlockSpec((H, 1), lambda i, j: (0, 0)),
                pl.BlockSpec((1, H), lambda i, j: (0, 0)),
                pl.BlockSpec((1, 1), lambda i, j: (0, 0)),
            ],
            out_specs=pl.BlockSpec((1, block_b),
                                   lambda i, j, g2=g2: (0, i * g2 + j)),
            scratch_shapes=[
                pltpu.VMEM((2, cin, block_b, L), jnp.float32),
                pltpu.SemaphoreType.DMA((2, cin)),
                pltpu.VMEM((R, block_b), jnp.bfloat16),
            ],
        ),
        compiler_params=pltpu.CompilerParams(
            dimension_semantics=("parallel", "arbitrary")),
    )(x, perm, wtoe, wtail, bconv2, whidT, bhid2, wneuT, bneu2)
    return out[0, :N].reshape(N, 1)


# speedup vs baseline: 1.0121x; 1.0121x over previous
import functools

import jax
import jax.numpy as jnp
from jax.experimental import pallas as pl
from jax.experimental.pallas import tpu as pltpu


def _fused_kernel(x_hbm, perm_ref, wtoe_ref, wtail_ref, bconv_ref,
                  whidT_ref, bhid_ref, wneuT_ref, bneu_ref, out_ref,
                  bufs, sems, x2_ref,
                  *, cin, m, p, s, nfull, tail_start, block_b):
    j = pl.program_id(1)
    g2 = pl.num_programs(1)
    base = (pl.program_id(0) * g2 + j) * block_b
    R = x2_ref.shape[0]
    B = block_b

    @pl.when(j == 0)
    def _():
        for c in range(cin):
            pltpu.make_async_copy(x_hbm.at[pl.ds(base, B), c],
                                  bufs.at[0, c], sems.at[0, c]).start()

    @pl.when(j + 1 < g2)
    def _():
        nbase = base + B
        nslot = (j + 1) % 2
        for c in range(cin):
            pltpu.make_async_copy(x_hbm.at[pl.ds(nbase, B), c],
                                  bufs.at[nslot, c], sems.at[nslot, c]).start()

    slot = j % 2
    for c in range(cin):
        pltpu.make_async_copy(bufs.at[slot, c], bufs.at[slot, c],
                              sems.at[slot, c]).wait()

    dn = (((1,), (1,)), ((), ()))
    xcb = [bufs[slot, c].astype(jnp.bfloat16) for c in range(cin)]
    for grp in range(R // 128):
        y = None
        for c in range(cin):
            pj = perm_ref[(grp * cin + c) * 128:(grp * cin + c + 1) * 128, :]
            d = jax.lax.dot_general(pj, xcb[c], dn,
                                    preferred_element_type=jnp.float32)
            y = d if y is None else y + d
        x2_ref[grp * 128:(grp + 1) * 128, :] = y.astype(jnp.bfloat16)

    wtoe = wtoe_ref[...]
    stride = p * cin
    feat = jnp.full((p * m, B), -1e30, jnp.float32)
    for c in range(nfull):
        xs = x2_ref[pl.ds(c * stride, s), :]
        feat = jnp.maximum(feat, jnp.dot(wtoe, xs,
                                         preferred_element_type=jnp.float32))
    if tail_start is not None:
        xs = x2_ref[pl.ds(tail_start, s), :]
        feat = jnp.maximum(feat, jnp.dot(wtail_ref[...], xs,
                                         preferred_element_type=jnp.float32))

    acc = feat[0:m, :]
    for q in range(1, p):
        acc = jnp.maximum(acc, feat[q * m:(q + 1) * m, :])
    acc = jnp.maximum(acc + bconv_ref[...], 0.0)

    h = jnp.dot(whidT_ref[...], acc,
                preferred_element_type=jnp.float32) + bhid_ref[...]
    h = jnp.maximum(h, 0.0)
    logit = jnp.dot(wneuT_ref[...], h,
                    preferred_element_type=jnp.float32) + bneu_ref[...]
    out_ref[...] = jax.nn.sigmoid(logit)


def _toeplitz(wflat, m, p, s, cin, shift):
    kc = wflat.shape[1]
    colidx = jnp.arange(s)[None, :] - jnp.arange(p)[:, None] * cin - shift
    valid = (colidx >= 0) & (colidx < kc)
    g = wflat[:, jnp.clip(colidx, 0, kc - 1)]
    wt = jnp.where(valid[None], g, 0.0)
    return wt.transpose(1, 0, 2).reshape(p * m, s).astype(jnp.bfloat16)


def kernel(x, wconv, bconv, whid, bhid, wneu, bneu, *, block_b=2048):
    N, cin, L = x.shape
    M, _, K = wconv.shape
    H = whid.shape[1]
    lout = L - K + 1
    P = 128 // M
    S = ((P - 1) * cin + cin * K + 127) // 128 * 128
    R = L * cin
    assert R % 128 == 0 and 128 % cin == 0
    nfull = lout // P
    assert nfull >= 1
    ntail = lout - nfull * P

    if ntail:
        l0 = lout - P
        tail_start = l0 * cin // 16 * 16
        shift = l0 * cin - tail_start
        assert shift + (P - 1) * cin + K * cin <= S
        assert tail_start + S <= R
    else:
        tail_start, shift = None, 0

    npad = max(2 * block_b, (N + 2 * block_b - 1) // (2 * block_b) * (2 * block_b))
    if npad != N:
        x = jnp.pad(x, ((0, npad - N), (0, 0), (0, 0)))

    ngrp = R // 128
    gidx = jnp.arange(ngrp * cin)[:, None]
    jj, cc = gidx // cin, gidx % cin
    r = jnp.arange(128)[None, :]
    tgt = jnp.where((r % cin) == cc, r // cin + (128 // cin) * jj, -1)
    perm = jax.nn.one_hot(tgt, L, dtype=jnp.float32).astype(
        jnp.bfloat16).reshape(ngrp * cin * 128, L)

    wflat = jnp.transpose(wconv.astype(jnp.float32), (0, 2, 1)).reshape(M, K * cin)
    wtoe = _toeplitz(wflat, M, P, S, cin, 0)
    wtail = _toeplitz(wflat, M, P, S, cin, shift) if ntail else wtoe

    bconv2 = bconv.reshape(M, 1).astype(jnp.float32)
    whidT = whid.T.astype(jnp.float32)
    bhid2 = bhid.reshape(H, 1).astype(jnp.float32)
    wneuT = wneu.T.astype(jnp.float32)
    bneu2 = bneu.reshape(1, 1).astype(jnp.float32)

    g2 = npad // (2 * block_b)
    kfn = functools.partial(_fused_kernel, cin=cin, m=M, p=P, s=S,
                            nfull=nfull, tail_start=tail_start,
                            block_b=block_b)
    out = pl.pallas_call(
        kfn,
        out_shape=jax.ShapeDtypeStruct((1, npad), jnp.float32),
        grid_spec=pltpu.PrefetchScalarGridSpec(
            num_scalar_prefetch=0,
            grid=(2, g2),
            in_specs=[
                pl.BlockSpec(memory_space=pl.ANY),
                pl.BlockSpec((ngrp * cin * 128, L), lambda i, j: (0, 0)),
                pl.BlockSpec((P * M, S), lambda i, j: (0, 0)),
                pl.BlockSpec((P * M, S), lambda i, j: (0, 0)),
                pl.BlockSpec((M, 1), lambda i, j: (0, 0)),
                pl.BlockSpec((H, M), lambda i, j: (0, 0)),
                pl.BlockSpec((H, 1), lambda i, j: (0, 0)),
                pl.BlockSpec((1, H), lambda i, j: (0, 0)),
                pl.BlockSpec((1, 1), lambda i, j: (0, 0)),
            ],
            out_specs=pl.BlockSpec((1, block_b),
                                   lambda i, j, g2=g2: (0, i * g2 + j)),
            scratch_shapes=[
                pltpu.VMEM((2, cin, block_b, L), jnp.float32),
                pltpu.SemaphoreType.DMA((2, cin)),
                pltpu.VMEM((R, block_b), jnp.bfloat16),
            ],
        ),
        compiler_params=pltpu.CompilerParams(
            dimension_semantics=("parallel", "arbitrary")),
    )(x, perm, wtoe, wtail, bconv2, whidT, bhid2, wneuT, bneu2)
    return out[0, :N].reshape(N, 1)


# speedup vs baseline: 1.0135x; 1.0014x over previous
import functools

import jax
import jax.numpy as jnp
from jax.experimental import pallas as pl
from jax.experimental.pallas import tpu as pltpu


_DEPTH = 3


def _fused_kernel(x_hbm, perm_ref, wt2_ref, bconv_ref,
                  whidT_ref, bhid_ref, wneuT_ref, bneu_ref, out_ref,
                  bufs, sems, x2_ref,
                  *, cin, m, p, s, nfull, tail_start, block_b):
    j = pl.program_id(1)
    g2 = pl.num_programs(1)
    base = (pl.program_id(0) * g2 + j) * block_b
    R = x2_ref.shape[0]
    B = block_b

    def issue(step):
        d = step % _DEPTH
        for c in range(cin):
            pltpu.make_async_copy(
                x_hbm.at[pl.ds((pl.program_id(0) * g2 + step) * B, B), c],
                bufs.at[d, c], sems.at[d, c]).start()

    @pl.when(j == 0)
    def _():
        for d in range(min(_DEPTH - 1, g2)):
            issue(d)

    @pl.when(j + _DEPTH - 1 < g2)
    def _():
        issue(j + _DEPTH - 1)

    slot = j % _DEPTH
    for c in range(cin):
        pltpu.make_async_copy(bufs.at[slot, c], bufs.at[slot, c],
                              sems.at[slot, c]).wait()

    dn = (((1,), (1,)), ((), ()))
    xcb = [bufs[slot, c].astype(jnp.bfloat16) for c in range(cin)]
    for grp in range(R // 128):
        y = None
        for c in range(cin):
            pj = perm_ref[(grp * cin + c) * 128:(grp * cin + c + 1) * 128, :]
            d = jax.lax.dot_general(pj, xcb[c], dn,
                                    preferred_element_type=jnp.float32)
            y = d if y is None else y + d
        x2_ref[grp * 128:(grp + 1) * 128, :] = y.astype(jnp.bfloat16)

    wtoe = wt2_ref[0:p * m, :]
    stride = p * cin
    feat = jnp.full((p * m, B), -1e30, jnp.float32)
    for c in range(nfull):
        xs = x2_ref[pl.ds(c * stride, s), :]
        feat = jnp.maximum(feat, jnp.dot(wtoe, xs,
                                         preferred_element_type=jnp.float32))
    if tail_start is not None:
        xs = x2_ref[pl.ds(tail_start, s), :]
        feat = jnp.maximum(feat, jnp.dot(wt2_ref[p * m:2 * p * m, :], xs,
                                         preferred_element_type=jnp.float32))

    acc = feat[0:m, :]
    for q in range(1, p):
        acc = jnp.maximum(acc, feat[q * m:(q + 1) * m, :])
    acc = jnp.maximum(acc + bconv_ref[...], 0.0)

    h = jnp.dot(whidT_ref[...], acc,
                preferred_element_type=jnp.float32) + bhid_ref[...]
    h = jnp.maximum(h, 0.0)
    logit = jnp.dot(wneuT_ref[...], h,
                    preferred_element_type=jnp.float32) + bneu_ref[...]
    out_ref[...] = jax.nn.sigmoid(logit)


def _toeplitz2(wflat, m, p, s, cin, shifts):
    kc = wflat.shape[1]
    sh = jnp.array(shifts)[:, None, None]
    colidx = (jnp.arange(s)[None, None, :]
              - jnp.arange(p)[None, :, None] * cin - sh)
    valid = (colidx >= 0) & (colidx < kc)
    g = wflat[:, jnp.clip(colidx, 0, kc - 1)]
    wt = jnp.where(valid[None], g, jnp.bfloat16(0.0))
    return wt.transpose(1, 2, 0, 3).reshape(len(shifts) * p * m, s)


def kernel(x, wconv, bconv, whid, bhid, wneu, bneu, *, block_b=2048):
    N, cin, L = x.shape
    M, _, K = wconv.shape
    H = whid.shape[1]
    lout = L - K + 1
    P = 128 // M
    S = ((P - 1) * cin + cin * K + 127) // 128 * 128
    R = L * cin
    assert R % 128 == 0 and 128 % cin == 0
    nfull = lout // P
    assert nfull >= 1
    ntail = lout - nfull * P

    if ntail:
        l0 = lout - P
        tail_start = l0 * cin // 16 * 16
        shift = l0 * cin - tail_start
        assert shift + (P - 1) * cin + K * cin <= S
        assert tail_start + S <= R
    else:
        tail_start, shift = None, 0

    npad = max(2 * block_b, (N + 2 * block_b - 1) // (2 * block_b) * (2 * block_b))
    if npad != N:
        x = jnp.pad(x, ((0, npad - N), (0, 0), (0, 0)))

    ngrp = R // 128
    gidx = jnp.arange(ngrp * cin)[:, None]
    jj, cc = gidx // cin, gidx % cin
    r = jnp.arange(128)[None, :]
    tgt = jnp.where((r % cin) == cc, r // cin + (128 // cin) * jj, -1)
    perm = jax.nn.one_hot(tgt, L, dtype=jnp.float32).astype(
        jnp.bfloat16).reshape(ngrp * cin * 128, L)

    wflat = jnp.transpose(wconv.astype(jnp.bfloat16),
                          (0, 2, 1)).reshape(M, K * cin)
    wt2 = _toeplitz2(wflat, M, P, S, cin, [0, shift])

    bconv2 = bconv.reshape(M, 1).astype(jnp.float32)
    whidT = whid.T.astype(jnp.float32)
    bhid2 = bhid.reshape(H, 1).astype(jnp.float32)
    wneuT = wneu.T.astype(jnp.float32)
    bneu2 = bneu.reshape(1, 1).astype(jnp.float32)

    g2 = npad // (2 * block_b)
    kfn = functools.partial(_fused_kernel, cin=cin, m=M, p=P, s=S,
                            nfull=nfull, tail_start=tail_start,
                            block_b=block_b)
    out = pl.pallas_call(
        kfn,
        out_shape=jax.ShapeDtypeStruct((1, npad), jnp.float32),
        grid_spec=pltpu.PrefetchScalarGridSpec(
            num_scalar_prefetch=0,
            grid=(2, g2),
            in_specs=[
                pl.BlockSpec(memory_space=pl.ANY),
                pl.BlockSpec((ngrp * cin * 128, L), lambda i, j: (0, 0)),
                pl.BlockSpec((2 * P * M, S), lambda i, j: (0, 0)),
                pl.BlockSpec((M, 1), lambda i, j: (0, 0)),
                pl.BlockSpec((H, M), lambda i, j: (0, 0)),
                pl.BlockSpec((H, 1), lambda i, j: (0, 0)),
                pl.BlockSpec((1, H), lambda i, j: (0, 0)),
                pl.BlockSpec((1, 1), lambda i, j: (0, 0)),
            ],
            out_specs=pl.BlockSpec((1, block_b),
                                   lambda i, j, g2=g2: (0, i * g2 + j)),
            scratch_shapes=[
                pltpu.VMEM((_DEPTH, cin, block_b, L), jnp.float32),
                pltpu.SemaphoreType.DMA((_DEPTH, cin)),
                pltpu.VMEM((R, block_b), jnp.bfloat16),
            ],
        ),
        compiler_params=pltpu.CompilerParams(
            dimension_semantics=("parallel", "arbitrary")),
    )(x, perm, wt2, bconv2, whidT, bhid2, wneuT, bneu2)
    return out[0, :N].reshape(N, 1)


# speedup vs baseline: 1.1699x; 1.1544x over previous
import functools

import jax
import jax.numpy as jnp
from jax.experimental import pallas as pl
from jax.experimental.pallas import tpu as pltpu

_DEPTH = 3


def _fused_kernel(x_hbm, wflat_ref, bconv_ref, whid_ref, bhid_ref,
                  wneu_ref, bneu_ref, out_ref,
                  bufs, sems, x2_ref, perm_ref, wt_ref,
                  *, cin, m, p, s, nfull, tail_start, tail_shift, block_b):
    j = pl.program_id(1)
    g2 = pl.num_programs(1)
    R = x2_ref.shape[0]
    B = block_b
    lgrp = 128 // cin
    kc = wflat_ref.shape[1]

    def issue(step):
        d = step % _DEPTH
        for c in range(cin):
            pltpu.make_async_copy(
                x_hbm.at[pl.ds((pl.program_id(0) * g2 + step) * B, B), c],
                bufs.at[d, c], sems.at[d, c]).start()

    @pl.when(j == 0)
    def _():
        for d in range(min(_DEPTH - 1, g2)):
            issue(d)
        rr = jax.lax.broadcasted_iota(jnp.int32, (128, 128), 0)
        ll = jax.lax.broadcasted_iota(jnp.int32, (128, 128), 1)
        for grp in range(R // 128):
            for c in range(cin):
                hit = ((rr % cin) == c) & (ll == rr // cin + lgrp * grp)
                perm_ref[(grp * cin + c) * 128:(grp * cin + c + 1) * 128, :] \
                    = jnp.where(hit, 1.0, 0.0).astype(jnp.bfloat16)
        aa = jax.lax.broadcasted_iota(jnp.int32, (kc, s), 0)
        bb = jax.lax.broadcasted_iota(jnp.int32, (kc, s), 1)
        wfl = wflat_ref[...]
        shifts = [0] if tail_start is None else [0, tail_shift]
        for t, sh in enumerate(shifts):
            for q in range(p):
                fq = jnp.where(bb == aa + q * cin + sh, 1.0,
                               0.0).astype(jnp.bfloat16)
                tmp = jnp.dot(wfl, fq, preferred_element_type=jnp.float32)
                wt_ref[t * p * m + q * m:t * p * m + (q + 1) * m, :] \
                    = tmp.astype(jnp.bfloat16)

    @pl.when(j + _DEPTH - 1 < g2)
    def _():
        issue(j + _DEPTH - 1)

    slot = j % _DEPTH
    for c in range(cin):
        pltpu.make_async_copy(bufs.at[slot, c], bufs.at[slot, c],
                              sems.at[slot, c]).wait()

    dn = (((1,), (1,)), ((), ()))
    d0 = (((0,), (0,)), ((), ()))
    xcb = [bufs[slot, c].astype(jnp.bfloat16) for c in range(cin)]
    for grp in range(R // 128):
        y = None
        for c in range(cin):
            pj = perm_ref[(grp * cin + c) * 128:(grp * cin + c + 1) * 128, :]
            d = jax.lax.dot_general(pj, xcb[c], dn,
                                    preferred_element_type=jnp.float32)
            y = d if y is None else y + d
        x2_ref[grp * 128:(grp + 1) * 128, :] = y.astype(jnp.bfloat16)

    wtoe = wt_ref[0:p * m, :]
    stride = p * cin
    feat = jnp.full((p * m, B), -1e30, jnp.float32)
    for c in range(nfull):
        xs = x2_ref[pl.ds(c * stride, s), :]
        feat = jnp.maximum(feat, jnp.dot(wtoe, xs,
                                         preferred_element_type=jnp.float32))
    if tail_start is not None:
        xs = x2_ref[pl.ds(tail_start, s), :]
        feat = jnp.maximum(feat, jnp.dot(wt_ref[p * m:2 * p * m, :], xs,
                                         preferred_element_type=jnp.float32))

    acc = feat[0:m, :]
    for q in range(1, p):
        acc = jnp.maximum(acc, feat[q * m:(q + 1) * m, :])
    acc = jnp.maximum(acc + bconv_ref[...], 0.0)

    h = jax.lax.dot_general(whid_ref[...], acc, d0,
                            preferred_element_type=jnp.float32) + bhid_ref[...]
    h = jnp.maximum(h, 0.0)
    logit = jax.lax.dot_general(wneu_ref[...], h, d0,
                                preferred_element_type=jnp.float32) \
        + bneu_ref[...]
    out_ref[...] = jax.nn.sigmoid(logit)


def kernel(x, wconv, bconv, whid, bhid, wneu, bneu, *, block_b=2048):
    N, cin, L = x.shape
    M, _, K = wconv.shape
    H = whid.shape[1]
    lout = L - K + 1
    P = 128 // M
    S = ((P - 1) * cin + cin * K + 127) // 128 * 128
    R = L * cin
    assert R % 128 == 0 and 128 % cin == 0 and L == 128
    nfull = lout // P
    assert nfull >= 1
    ntail = lout - nfull * P

    if ntail:
        l0 = lout - P
        tail_start = l0 * cin // 16 * 16
        tail_shift = l0 * cin - tail_start
        assert tail_shift + (P - 1) * cin + K * cin <= S
        assert tail_start + S <= R
    else:
        tail_start, tail_shift = None, 0

    npad = max(2 * block_b, (N + 2 * block_b - 1) // (2 * block_b) * (2 * block_b))
    if npad != N:
        x = jnp.pad(x, ((0, npad - N), (0, 0), (0, 0)))

    wflat = jnp.transpose(wconv.astype(jnp.bfloat16),
                          (0, 2, 1)).reshape(M, K * cin)
    bconv2 = bconv.reshape(M, 1).astype(jnp.float32)
    bhid2 = bhid.reshape(H, 1).astype(jnp.float32)
    bneu2 = bneu.reshape(1, 1).astype(jnp.float32)

    g2 = npad // (2 * block_b)
    kfn = functools.partial(_fused_kernel, cin=cin, m=M, p=P, s=S,
                            nfull=nfull, tail_start=tail_start,
                            tail_shift=tail_shift, block_b=block_b)
    out = pl.pallas_call(
        kfn,
        out_shape=jax.ShapeDtypeStruct((1, npad), jnp.float32),
        grid_spec=pltpu.PrefetchScalarGridSpec(
            num_scalar_prefetch=0,
            grid=(2, g2),
            in_specs=[
                pl.BlockSpec(memory_space=pl.ANY),
                pl.BlockSpec((M, K * cin), lambda i, j: (0, 0)),
                pl.BlockSpec((M, 1), lambda i, j: (0, 0)),
                pl.BlockSpec((M, H), lambda i, j: (0, 0)),
                pl.BlockSpec((H, 1), lambda i, j: (0, 0)),
                pl.BlockSpec((H, 1), lambda i, j: (0, 0)),
                pl.BlockSpec((1, 1), lambda i, j: (0, 0)),
            ],
            out_specs=pl.BlockSpec((1, block_b),
                                   lambda i, j, g2=g2: (0, i * g2 + j)),
            scratch_shapes=[
                pltpu.VMEM((_DEPTH, cin, block_b, L), jnp.float32),
                pltpu.SemaphoreType.DMA((_DEPTH, cin)),
                pltpu.VMEM((R, block_b), jnp.bfloat16),
                pltpu.VMEM((R // 128 * cin * 128, 128), jnp.bfloat16),
                pltpu.VMEM((2 * P * M, S), jnp.bfloat16),
            ],
        ),
        compiler_params=pltpu.CompilerParams(
            dimension_semantics=("parallel", "arbitrary")),
    )(x, wflat, bconv2, whid, bhid2, wneu, bneu2)
    return out[0, :N].reshape(N, 1)
